# Initial kernel scaffold; baseline (speedup 1.0000x reference)
import jax
import jax.numpy as jnp
from jax.experimental import pallas as pl


def kernel(x, Wr, br, Wn, bn):
    raise NotImplementedError("write your pallas kernel here")


# speedup vs baseline: 4.8880x; 4.8880x over previous
import functools

import jax
import jax.numpy as jnp
from jax import lax
from jax.experimental import pallas as pl

N = 16384
EMB = 4096
E = 64
K = 8
BLOCK = 512


def _router_block(x_ref, wt_ref, br_ref, scores_ref, idx_ref):
    logits = jnp.dot(x_ref[...], wt_ref[...], preferred_element_type=jnp.float32)
    logits = logits + br_ref[...]

    col = lax.broadcasted_iota(jnp.int32, (BLOCK, E), 1)
    neg_inf = jnp.float32(-jnp.inf)
    cur = logits
    top_vals = []
    idx_cols = []
    for _ in range(K):
        m = jnp.max(cur, axis=1, keepdims=True)
        amax = jnp.min(jnp.where(cur == m, col, E), axis=1, keepdims=True)
        top_vals.append(m)
        idx_cols.append(amax)
        cur = jnp.where(col == amax, neg_inf, cur)

    row_max = top_vals[0]
    denom = jnp.zeros_like(row_max)
    for v in top_vals:
        denom = denom + jnp.exp(v - row_max)
    selected = cur == neg_inf
    scores_ref[...] = jnp.where(
        selected, jnp.exp(logits - row_max) / denom, jnp.float32(0.0)
    )
    idx_ref[...] = jnp.concatenate(idx_cols, axis=1)


def kernel(x, Wr, br, Wn, bn):
    del Wn, bn
    wt = Wr.T
    brow = br.reshape(1, E)
    grid = (N // BLOCK,)
    scores, idx = pl.pallas_call(
        _router_block,
        grid=grid,
        in_specs=[
            pl.BlockSpec((BLOCK, EMB), lambda i: (i, 0)),
            pl.BlockSpec((EMB, E), lambda i: (0, 0)),
            pl.BlockSpec((1, E), lambda i: (0, 0)),
        ],
        out_specs=[
            pl.BlockSpec((BLOCK, E), lambda i: (i, 0)),
            pl.BlockSpec((BLOCK, K), lambda i: (i, 0)),
        ],
        out_shape=[
            jax.ShapeDtypeStruct((N, E), jnp.float32),
            jax.ShapeDtypeStruct((N, K), jnp.int32),
        ],
    )(x, wt, brow)
    return scores, idx
